# baseline (speedup 1.0000x reference)
import functools

import jax
import jax.numpy as jnp
from jax import lax
from jax.experimental import pallas as pl
from jax.experimental.pallas import tpu as pltpu
from jax.experimental.pallas import tpu_sc as plsc

_BATCH = 16384
_DIM = 64
_NC = 2
_NS = 16
_NW = _NC * _NS
_BPW = _BATCH // _NW
_CHUNK = 128
_NCHUNK = _BPW // _CHUNK


def _gather_body(table_hbm, idx_hbm, out_hbm, idx_v, rows_v, sem):
    wid = lax.axis_index("s") * _NC + lax.axis_index("c")
    base = wid * _BPW
    for j in range(_NCHUNK):
        pltpu.sync_copy(idx_hbm.at[pl.ds(base + j * _CHUNK, _CHUNK)], idx_v.at[j])
    copies = [
        pltpu.async_copy(
            table_hbm.at[idx_v.at[j]],
            rows_v.at[pl.ds(j * _CHUNK, _CHUNK)],
            sem,
        )
        for j in range(_NCHUNK)
    ]
    for c in copies:
        c.wait()
    pltpu.sync_copy(rows_v, out_hbm.at[pl.ds(base, _BPW)])


@jax.jit
def kernel(idx, latent_codes):
    run = pl.kernel(
        _gather_body,
        mesh=plsc.VectorSubcoreMesh(core_axis_name="c", subcore_axis_name="s"),
        out_type=jax.ShapeDtypeStruct((_BATCH, _DIM), jnp.float32),
        scratch_types=[
            pltpu.VMEM((_NCHUNK, _CHUNK), jnp.int32),
            pltpu.VMEM((_BPW, _DIM), jnp.float32),
            pltpu.SemaphoreType.DMA,
        ],
        compiler_params=pltpu.CompilerParams(use_tc_tiling_on_sc=False),
    )
    return run(latent_codes, idx.astype(jnp.int32))


# speedup vs baseline: 2.5674x; 2.5674x over previous
import jax
import jax.numpy as jnp
from jax import lax
from jax.experimental import pallas as pl
from jax.experimental.pallas import tpu as pltpu
from jax.experimental.pallas import tpu_sc as plsc

_BATCH = 16384
_DIM = 64
_NC = 2
_NS = 16
_NW = _NC * _NS
_BPW = _BATCH // _NW
_OBLK = _BPW // 8


def _gather_body(table_hbm, idx_hbm, out_hbm, idx_v, idx_s, out_v, sem):
    wid = lax.axis_index("s") * _NC + lax.axis_index("c")
    base = wid * _BPW

    pltpu.sync_copy(idx_hbm.at[pl.ds(base, _BPW)], idx_v)

    def fire(g, _):
        vec = idx_v[pl.ds(g * 16, 16)]
        r0 = g * 16
        for j in range(16):
            v = vec[j]
            b = lax.shift_right_logical(v, 3)
            s = lax.rem(v, jnp.int32(8))
            r = r0 + j
            pltpu.async_copy(
                table_hbm.at[b, s],
                out_v.at[lax.div(r, jnp.int32(8)), lax.rem(r, jnp.int32(8))],
                sem,
            )
        return 0

    lax.fori_loop(0, _BPW // 16, fire, 0)

    def drain(r, _):
        pltpu.make_async_copy(table_hbm.at[0, 0], out_v.at[0, 0], sem).wait()
        return 0

    lax.fori_loop(0, _BPW, drain, 0, unroll=8)

    pltpu.sync_copy(out_v, out_hbm.at[pl.ds(wid * _OBLK, _OBLK)])


@jax.jit
def kernel(idx, latent_codes):
    table3 = latent_codes.reshape(125000, 8, _DIM)
    run = pl.kernel(
        _gather_body,
        mesh=plsc.VectorSubcoreMesh(core_axis_name="c", subcore_axis_name="s"),
        out_type=jax.ShapeDtypeStruct((_BATCH // 8, 8, _DIM), jnp.float32),
        scratch_types=[
            pltpu.VMEM((_BPW,), jnp.int32),
            pltpu.SMEM((_BPW,), jnp.int32),
            pltpu.VMEM((_OBLK, 8, _DIM), jnp.float32),
            pltpu.SemaphoreType.DMA,
        ],
        compiler_params=pltpu.CompilerParams(use_tc_tiling_on_sc=True),
    )
    out3 = run(table3, idx.astype(jnp.int32))
    return out3.reshape(_BATCH, _DIM)
